# baseline (speedup 1.0000x reference)
import functools

import jax
import jax.numpy as jnp
from jax import lax
from jax.experimental import pallas as pl
from jax.experimental.pallas import tpu as pltpu
from jax.experimental.pallas import tpu_sc as plsc

N = 100000
E = 1600000
H = 128
G = 64

N_TC = 100352
NBLOCKS = 98
BLK = 1024
NC_ROWS = 12544
STRIPE = 784
NPASS = 4
E_PAD = 1601536
EPT = 100096
EBLK = 128
NEBLK = EPT // EBLK


def _sc_body(x_ref, src_ref, dst_ref, a_ref, w_ref, be_ref, agg_ref,
             agg_sp, dblk, sblk, ablk, gat_i, gat_d, xg, wv, bev, sem):
    c = lax.axis_index("c")
    s = lax.axis_index("s")
    t0 = s * EPT
    pltpu.sync_copy(w_ref, wv)
    pltpu.sync_copy(be_ref, bev)
    iota16 = lax.iota(jnp.int32, 16)
    wregs = [wv[pl.ds(16 * j, 16)] for j in range(8)]
    bregs = [bev[pl.ds(16 * j, 16)] for j in range(8)]

    def row_body(g, _):
        a16 = ablk[pl.ds(16 * g, 16)]
        for k in range(16):
            i = 16 * g + k
            asp = jnp.full((16,), a16[k])
            for j in range(8):
                xv = xg[i, pl.ds(16 * j, 16)]
                xg[i, pl.ds(16 * j, 16)] = jnp.maximum(
                    xv + asp * wregs[j] + bregs[j], 0.0)
        return 0

    def blk_body(b, carry, base):
        off = t0 + EBLK * b
        pltpu.sync_copy(dst_ref.at[pl.ds(off, EBLK)], dblk)
        pltpu.sync_copy(src_ref.at[pl.ds(off, EBLK)], sblk)
        pltpu.sync_copy(a_ref.at[pl.ds(off, EBLK)], ablk)
        for v in range(EBLK // 16):
            dl = dblk[pl.ds(16 * v, 16)] - base
            msk = (dl >= 0) & (dl < NC_ROWS)
            gat_d[pl.ds(16 * v, 16)] = jnp.where(msk, dl, NC_ROWS + iota16)
            gat_i[pl.ds(16 * v, 16)] = jnp.where(
                msk, sblk[pl.ds(16 * v, 16)], iota16)
        pltpu.async_copy(x_ref.at[gat_i], xg, sem).wait()
        lax.fori_loop(0, EBLK // 16, row_body, 0)
        pltpu.sync_copy(xg, agg_sp.at[gat_d], add=True)
        return carry

    def zero_row(i, _):
        for j in range(8):
            xg[i, pl.ds(16 * j, 16)] = jnp.zeros((16,), jnp.float32)
        return 0

    for p in range(NPASS):
        chunk = 2 * p + c
        base = chunk * NC_ROWS
        lax.fori_loop(0, EBLK, zero_row, 0)
        r0 = s * STRIPE
        for k in range(6):
            pltpu.sync_copy(xg, agg_sp.at[pl.ds(r0 + 128 * k, 128)])
        pltpu.sync_copy(xg.at[pl.ds(0, 16)], agg_sp.at[pl.ds(r0 + 768, 16)])
        plsc.subcore_barrier()
        lax.fori_loop(0, NEBLK, lambda b, cn: blk_body(b, cn, base),
                      jnp.int32(0))
        plsc.subcore_barrier()
        g0 = base + r0
        for k in range(6):
            pltpu.sync_copy(agg_sp.at[pl.ds(r0 + 128 * k, 128)],
                            agg_ref.at[pl.ds(g0 + 128 * k, 128)])
        pltpu.sync_copy(agg_sp.at[pl.ds(r0 + 768, 16)],
                        agg_ref.at[pl.ds(g0 + 768, 16)])
        plsc.subcore_barrier()


@functools.partial(
    pl.kernel,
    out_type=jax.ShapeDtypeStruct((N_TC, H), jnp.float32),
    mesh=plsc.VectorSubcoreMesh(core_axis_name="c", subcore_axis_name="s"),
    scratch_types=[
        pltpu.VMEM_SHARED((NC_ROWS + 16, H), jnp.float32),
        pltpu.VMEM((EBLK,), jnp.int32),
        pltpu.VMEM((EBLK,), jnp.int32),
        pltpu.VMEM((EBLK,), jnp.float32),
        pltpu.VMEM((EBLK,), jnp.int32),
        pltpu.VMEM((EBLK,), jnp.int32),
        pltpu.VMEM((EBLK, H), jnp.float32),
        pltpu.VMEM((H,), jnp.float32),
        pltpu.VMEM((H,), jnp.float32),
        pltpu.SemaphoreType.DMA,
    ],
)
def _sc_message(x, src, dst, a, w, be, agg,
                agg_sp, dblk, sblk, ablk, gat_i, gat_d, xg, wv, bev, sem):
    _sc_body(x, src, dst, a, w, be, agg,
             agg_sp, dblk, sblk, ablk, gat_i, gat_d, xg, wv, bev, sem)


def _proj_body(h_ref, w_ref, b_ref, o_ref):
    o_ref[...] = jnp.maximum(
        jnp.dot(h_ref[...], w_ref[...], preferred_element_type=jnp.float32)
        + b_ref[...], 0.0)


def _conv_body(x_ref, agg_ref, wn_ref, bn_ref, y_ref, cs_ref, ss_ref):
    i = pl.program_id(0)
    t = x_ref[...] + agg_ref[...]
    y = jnp.maximum(
        jnp.dot(t, wn_ref[...], preferred_element_type=jnp.float32)
        + bn_ref[...], 0.0)
    y_ref[...] = y
    rows = i * BLK + lax.broadcasted_iota(jnp.int32, (BLK, 1), 0)
    ym = jnp.where(rows < N, y, 0.0)

    @pl.when(i == 0)
    def _():
        cs_ref[...] = jnp.zeros_like(cs_ref)
        ss_ref[...] = jnp.zeros_like(ss_ref)

    cs_ref[...] += jnp.sum(ym, axis=0, keepdims=True)
    ss_ref[...] += jnp.sum(ym * ym, axis=0, keepdims=True)


def _norm_body(y_ref, cs_ref, ss_ref, o_ref):
    mu = cs_ref[...] / float(N)
    var = jnp.sum(ss_ref[...]) / float(N) - jnp.sum(mu * mu)
    denom = 1e-5 + jnp.sqrt(jnp.maximum(var, 0.0))
    o_ref[...] = (y_ref[...] - mu) / denom


def _pool_body(y_ref, bt_ref, w1_ref, b1_ref, w2_ref, b2_ref, o_ref, hg_ref):
    i = pl.program_id(0)

    @pl.when(i == 0)
    def _():
        hg_ref[...] = jnp.full_like(hg_ref, -jnp.inf)

    bt = bt_ref[...].reshape(BLK, 1)
    y = y_ref[...]
    glo = jnp.min(bt)
    ghi = jnp.minimum(jnp.max(bt), G - 1)

    def gbody(g, _):
        m = bt == g
        contrib = jnp.max(jnp.where(m, y, -jnp.inf), axis=0, keepdims=True)
        hg_ref[pl.ds(g, 1), :] = jnp.maximum(hg_ref[pl.ds(g, 1), :], contrib)
        return 0

    lax.fori_loop(glo, ghi + 1, gbody, 0)

    @pl.when(i == NBLOCKS - 1)
    def _():
        t = jnp.maximum(
            jnp.dot(hg_ref[...], w1_ref[...],
                    preferred_element_type=jnp.float32) + b1_ref[...], 0.0)
        o_ref[...] = jnp.dot(t, w2_ref[...],
                             preferred_element_type=jnp.float32) + b2_ref[...]


def _full(shape):
    return pl.BlockSpec(shape, lambda i: tuple(0 for _ in shape))


def kernel(h, edge_index, edge_attr, batch, W_in, b_in,
           We1, be1, Wn1, bn1, We2, be2, Wn2, bn2, We3, be3, Wn3, bn3,
           W1, b1, W2, b2):
    f32 = jnp.float32
    h_p = jnp.zeros((N_TC, 8), f32).at[:N, :5].set(h)
    W_inp = jnp.zeros((8, H), f32).at[:5].set(W_in)
    src_p = jnp.concatenate([edge_index[0], jnp.zeros((E_PAD - E,), jnp.int32)])
    dst_p = jnp.concatenate([edge_index[1],
                             jnp.full((E_PAD - E,), N_TC - 1, jnp.int32)])
    a_p = jnp.concatenate([edge_attr[:, 0], jnp.zeros((E_PAD - E,), f32)])
    batch_r = jnp.concatenate(
        [batch, jnp.full((N_TC - N,), G, jnp.int32)]).reshape(NBLOCKS, BLK, 1)
    W2p = jnp.zeros((H, 128), f32).at[:, :2].set(W2)
    b2p = jnp.zeros((1, 128), f32).at[0, :2].set(b2)

    x = pl.pallas_call(
        _proj_body,
        grid=(NBLOCKS,),
        in_specs=[pl.BlockSpec((BLK, 8), lambda i: (i, 0)),
                  _full((8, H)), _full((1, H))],
        out_specs=pl.BlockSpec((BLK, H), lambda i: (i, 0)),
        out_shape=jax.ShapeDtypeStruct((N_TC, H), f32),
    )(h_p, W_inp, b_in.reshape(1, H))

    for We, be, Wn, bn in ((We1, be1, Wn1, bn1), (We2, be2, Wn2, bn2),
                           (We3, be3, Wn3, bn3)):
        agg = _sc_message(x, src_p, dst_p, a_p, We[0], be)
        y, cs, ss = pl.pallas_call(
            _conv_body,
            grid=(NBLOCKS,),
            in_specs=[pl.BlockSpec((BLK, H), lambda i: (i, 0)),
                      pl.BlockSpec((BLK, H), lambda i: (i, 0)),
                      _full((H, H)), _full((1, H))],
            out_specs=[pl.BlockSpec((BLK, H), lambda i: (i, 0)),
                       _full((1, H)), _full((1, H))],
            out_shape=[jax.ShapeDtypeStruct((N_TC, H), f32),
                       jax.ShapeDtypeStruct((1, H), f32),
                       jax.ShapeDtypeStruct((1, H), f32)],
        )(x, agg, Wn, bn.reshape(1, H))
        x = pl.pallas_call(
            _norm_body,
            grid=(NBLOCKS,),
            in_specs=[pl.BlockSpec((BLK, H), lambda i: (i, 0)),
                      _full((1, H)), _full((1, H))],
            out_specs=pl.BlockSpec((BLK, H), lambda i: (i, 0)),
            out_shape=jax.ShapeDtypeStruct((N_TC, H), f32),
        )(y, cs, ss)

    out = pl.pallas_call(
        _pool_body,
        grid=(NBLOCKS,),
        in_specs=[pl.BlockSpec((BLK, H), lambda i: (i, 0)),
                  pl.BlockSpec((1, BLK, 1), lambda i: (i, 0, 0)),
                  _full((H, H)), _full((1, H)), _full((H, 128)),
                  _full((1, 128))],
        out_specs=_full((G, 128)),
        out_shape=jax.ShapeDtypeStruct((G, 128), f32),
        scratch_shapes=[pltpu.VMEM((G, 128), f32)],
    )(x, batch_r, W1, b1.reshape(1, H), W2p, b2p)
    return out[:, :2]


# speedup vs baseline: 11.2096x; 11.2096x over previous
import functools

import jax
import jax.numpy as jnp
from jax import lax
from jax.experimental import pallas as pl
from jax.experimental.pallas import tpu as pltpu
from jax.experimental.pallas import tpu_sc as plsc

N = 100000
E = 1600000
H = 128
G = 64

N_TC = 100352
NBLOCKS = 98
BLK = 1024
NC_ROWS = 11904
STRIPE = 744
NCH = 9
NPASS = 5
AGG_ROWS = 107520
E_PAD = 1601536
REG_E = 50048
SLOT = 50176
E_PAD2 = 32 * SLOT + 384
PBLK = 128
NPBLK = REG_E // PBLK


def _chunk_of(d):
    ch = jnp.zeros((16,), jnp.int32)
    for k in range(1, NCH):
        ch = ch + jnp.where(d >= k * NC_ROWS, 1, 0)
    return ch


def _part_body(src_ref, dst_ref, a_ref, src2_ref, dl2_ref, a2_ref,
               starts_ref, lens_ref,
               dblk, sblk, ablk, stg_s, stg_d, stg_a, cvec,
               fill, done, sstart, sem):
    c = lax.axis_index("c")
    s = lax.axis_index("s")
    w = 2 * s + c
    bin_ = w * REG_E
    bout = w * SLOT
    iota16 = lax.iota(jnp.int32, 16)

    def cnt_blk(b, accs):
        pltpu.sync_copy(dst_ref.at[pl.ds(bin_ + PBLK * b, PBLK)], dblk)

        def cnt_vreg(v, acc2):
            ch = _chunk_of(dblk[pl.ds(16 * v, 16)])
            return tuple(acc2[cc] + jnp.where(ch == cc, 1, 0)
                         for cc in range(NCH))

        return lax.fori_loop(0, PBLK // 16, cnt_vreg, accs)

    accs = lax.fori_loop(
        0, NPBLK, cnt_blk,
        tuple(jnp.zeros((16,), jnp.int32) for _ in range(NCH)))

    run = jnp.int32(0)
    startvec = jnp.zeros((16,), jnp.int32)
    lenvec = jnp.zeros((16,), jnp.int32)
    for cc in range(NCH):
        tot = accs[cc][0]
        for k in range(1, 16):
            tot = tot + accs[cc][k]
        plc = jnp.bitwise_and(tot + 7, -8)
        startvec = jnp.where(iota16 == cc, run, startvec)
        lenvec = jnp.where(iota16 == cc, plc, lenvec)
        sstart[cc] = run
        fill[cc] = jnp.int32(0)
        done[cc] = jnp.int32(0)
        run = run + plc
    cvec[...] = startvec
    pltpu.sync_copy(cvec, starts_ref.at[w])
    cvec[...] = lenvec
    pltpu.sync_copy(cvec, lens_ref.at[w])

    def sc_blk(b, _):
        off = bin_ + PBLK * b
        pltpu.sync_copy(dst_ref.at[pl.ds(off, PBLK)], dblk)
        pltpu.sync_copy(src_ref.at[pl.ds(off, PBLK)], sblk)
        pltpu.sync_copy(a_ref.at[pl.ds(off, PBLK)], ablk)

        def sc_vreg(v, _2):
            d16 = dblk[pl.ds(16 * v, 16)]
            s16 = sblk[pl.ds(16 * v, 16)]
            a16 = ablk[pl.ds(16 * v, 16)]
            ch16 = _chunk_of(d16)
            dl16 = d16 - ch16 * NC_ROWS
            for k in range(16):
                ck = ch16[k]
                n = fill[ck]
                al = jnp.bitwise_and(n, -16)
                msel = iota16 == (n - al)
                vd = stg_d[ck, pl.ds(al, 16)]
                stg_d[ck, pl.ds(al, 16)] = jnp.where(msel, dl16[k], vd)
                vs = stg_s[ck, pl.ds(al, 16)]
                stg_s[ck, pl.ds(al, 16)] = jnp.where(msel, s16[k], vs)
                va = stg_a[ck, pl.ds(al, 16)]
                stg_a[ck, pl.ds(al, 16)] = jnp.where(msel, a16[k], va)

                def flsh(z, ck=ck, n=n):
                    pos = pl.multiple_of(bout + sstart[ck] + done[ck], 8)
                    pltpu.sync_copy(stg_d.at[ck], dl2_ref.at[pl.ds(pos, 64)])
                    pltpu.sync_copy(stg_s.at[ck], src2_ref.at[pl.ds(pos, 64)])
                    pltpu.sync_copy(stg_a.at[ck], a2_ref.at[pl.ds(pos, 64)])
                    done[ck] = done[ck] + 64
                    fill[ck] = jnp.int32(0)
                    return z

                def noflsh(z, ck=ck, n=n):
                    fill[ck] = n + 1
                    return z

                lax.cond(n + 1 >= 64, flsh, noflsh, 0)
            return 0

        lax.fori_loop(0, PBLK // 16, sc_vreg, 0)
        return 0

    lax.fori_loop(0, NPBLK, sc_blk, 0)

    for ck in range(NCH):
        def pad1(q, _, ck=ck):
            n = fill[ck]
            al = jnp.bitwise_and(n, -16)
            ph = n - al
            msel = iota16 == ph
            vd = stg_d[ck, pl.ds(al, 16)]
            stg_d[ck, pl.ds(al, 16)] = jnp.where(msel, NC_ROWS + ph, vd)
            vs = stg_s[ck, pl.ds(al, 16)]
            stg_s[ck, pl.ds(al, 16)] = jnp.where(msel, ph, vs)
            va = stg_a[ck, pl.ds(al, 16)]
            stg_a[ck, pl.ds(al, 16)] = jnp.where(msel, 0.0, va)
            fill[ck] = n + 1
            return 0

        lax.fori_loop(0, jnp.bitwise_and(-fill[ck], 7), pad1, 0)

        def fl8(q, _, ck=ck):
            o8 = 8 * q
            pos = pl.multiple_of(bout + sstart[ck] + done[ck] + o8, 8)
            pltpu.sync_copy(stg_d.at[ck, pl.ds(o8, 8)],
                            dl2_ref.at[pl.ds(pos, 8)])
            pltpu.sync_copy(stg_s.at[ck, pl.ds(o8, 8)],
                            src2_ref.at[pl.ds(pos, 8)])
            pltpu.sync_copy(stg_a.at[ck, pl.ds(o8, 8)],
                            a2_ref.at[pl.ds(pos, 8)])
            return 0

        lax.fori_loop(0, fill[ck] // 8, fl8, 0)


@functools.partial(
    pl.kernel,
    out_type=(jax.ShapeDtypeStruct((E_PAD2,), jnp.int32),
              jax.ShapeDtypeStruct((E_PAD2,), jnp.int32),
              jax.ShapeDtypeStruct((E_PAD2,), jnp.float32),
              jax.ShapeDtypeStruct((32, 16), jnp.int32),
              jax.ShapeDtypeStruct((32, 16), jnp.int32)),
    mesh=plsc.VectorSubcoreMesh(core_axis_name="c", subcore_axis_name="s"),
    scratch_types=[
        pltpu.VMEM((PBLK,), jnp.int32),
        pltpu.VMEM((PBLK,), jnp.int32),
        pltpu.VMEM((PBLK,), jnp.float32),
        pltpu.VMEM((NCH, 64), jnp.int32),
        pltpu.VMEM((NCH, 64), jnp.int32),
        pltpu.VMEM((NCH, 64), jnp.float32),
        pltpu.VMEM((16,), jnp.int32),
        pltpu.SMEM((16,), jnp.int32),
        pltpu.SMEM((16,), jnp.int32),
        pltpu.SMEM((16,), jnp.int32),
        pltpu.SemaphoreType.DMA,
    ],
)
def _partition(src, dst, a, src2, dl2, a2, starts, lens,
               dblk, sblk, ablk, stg_s, stg_d, stg_a, cvec,
               fill, done, sstart, sem):
    _part_body(src, dst, a, src2, dl2, a2, starts, lens,
               dblk, sblk, ablk, stg_s, stg_d, stg_a, cvec,
               fill, done, sstart, sem)


def _msg_body(x_ref, s2_ref, d2_ref, a2_ref, st_ref, ln_ref, w_ref, be_ref,
              agg_ref, agg_sp, stv, lnv, gatiA, gatdA, gatiB, gatdB,
              ablkA, ablkB, dblk, sblk, xgA, xgB, wv, bev, semA, semB):
    c = lax.axis_index("c")
    s = lax.axis_index("s")
    pltpu.sync_copy(w_ref, wv)
    pltpu.sync_copy(be_ref, bev)
    iota16 = lax.iota(jnp.int32, 16)
    wregs = [wv[pl.ds(16 * j, 16)] for j in range(8)]
    bregs = [bev[pl.ds(16 * j, 16)] for j in range(8)]

    def mk_rows(xg, ablk):
        def row_body(g, _):
            a16 = ablk[pl.ds(16 * g, 16)]
            for k in range(16):
                i = 16 * g + k
                asp = jnp.full((16,), a16[k])
                for j in range(8):
                    xv = xg[i, pl.ds(16 * j, 16)]
                    xg[i, pl.ds(16 * j, 16)] = jnp.maximum(
                        xv + asp * wregs[j] + bregs[j], 0.0)
            return 0
        return row_body

    rowA = mk_rows(xgA, ablkA)
    rowB = mk_rows(xgB, ablkB)

    def prep(boff, seg0, seglen, gi, gd, ab):
        off = pl.multiple_of(seg0 + PBLK * boff, 8)
        pltpu.sync_copy(d2_ref.at[pl.ds(off, PBLK)], dblk)
        pltpu.sync_copy(s2_ref.at[pl.ds(off, PBLK)], sblk)
        pltpu.sync_copy(a2_ref.at[pl.ds(off, PBLK)], ab)
        lpos = PBLK * boff
        for v in range(PBLK // 16):
            msk = (lpos + 16 * v + iota16) < seglen
            gd[pl.ds(16 * v, 16)] = jnp.where(
                msk, dblk[pl.ds(16 * v, 16)], NC_ROWS + iota16)
            gi[pl.ds(16 * v, 16)] = jnp.where(
                msk, sblk[pl.ds(16 * v, 16)], iota16)

    def seg_process(seg0, seglen):
        npair = (seglen + 2 * PBLK - 1) // (2 * PBLK)
        prep(0, seg0, seglen, gatiA, gatdA, ablkA)
        pltpu.async_copy(x_ref.at[gatiA], xgA, semA)

        def pair(k2, _):
            b = 2 * k2
            prep(b + 1, seg0, seglen, gatiB, gatdB, ablkB)
            pltpu.async_copy(x_ref.at[gatiB], xgB, semB)
            pltpu.make_async_copy(x_ref.at[gatiA], xgA, semA).wait()
            lax.fori_loop(0, PBLK // 16, rowA, 0)
            pltpu.sync_copy(xgA, agg_sp.at[gatdA], add=True)
            prep(b + 2, seg0, seglen, gatiA, gatdA, ablkA)
            pltpu.async_copy(x_ref.at[gatiA], xgA, semA)
            pltpu.make_async_copy(x_ref.at[gatiB], xgB, semB).wait()
            lax.fori_loop(0, PBLK // 16, rowB, 0)
            pltpu.sync_copy(xgB, agg_sp.at[gatdB], add=True)
            return 0

        lax.fori_loop(0, npair, pair, 0)
        pltpu.make_async_copy(x_ref.at[gatiA], xgA, semA).wait()

    def zero_row(i, _):
        for j in range(8):
            xgA[i, pl.ds(16 * j, 16)] = jnp.zeros((16,), jnp.float32)
        return 0

    def pass_body(p, _):
        chunk = 2 * p + c
        valid = chunk <= NCH - 1
        r0 = s * STRIPE

        @pl.when(valid)
        def _():
            lax.fori_loop(0, PBLK, zero_row, 0)
            for k in range(5):
                pltpu.sync_copy(xgA, agg_sp.at[pl.ds(r0 + 128 * k, 128)])
            pltpu.sync_copy(xgA.at[pl.ds(0, 104)],
                            agg_sp.at[pl.ds(r0 + 640, 104)])

        plsc.subcore_barrier()

        @pl.when(valid)
        def _():
            def reg_body(r, _2):
                w2 = 2 * s + r
                pltpu.sync_copy(st_ref.at[w2], stv)
                pltpu.sync_copy(ln_ref.at[w2], lnv)
                svec = stv[pl.ds(0, 16)]
                lvec = lnv[pl.ds(0, 16)]
                seg_start = jnp.int32(0)
                seg_len = jnp.int32(0)
                for q in range(NCH):
                    seg_start = jnp.where(chunk == q, svec[q], seg_start)
                    seg_len = jnp.where(chunk == q, lvec[q], seg_len)
                seg_process(pl.multiple_of(w2 * SLOT + seg_start, 8),
                            seg_len)
                return 0

            lax.fori_loop(0, 2, reg_body, 0)

        plsc.subcore_barrier()

        @pl.when(valid)
        def _():
            g0 = chunk * NC_ROWS + r0
            for k in range(5):
                pltpu.sync_copy(agg_sp.at[pl.ds(r0 + 128 * k, 128)],
                                agg_ref.at[pl.ds(g0 + 128 * k, 128)])
            pltpu.sync_copy(agg_sp.at[pl.ds(r0 + 640, 104)],
                            agg_ref.at[pl.ds(g0 + 640, 104)])

        plsc.subcore_barrier()
        return 0

    lax.fori_loop(0, NPASS, pass_body, 0)


@functools.partial(
    pl.kernel,
    out_type=jax.ShapeDtypeStruct((AGG_ROWS, H), jnp.float32),
    mesh=plsc.VectorSubcoreMesh(core_axis_name="c", subcore_axis_name="s"),
    scratch_types=[
        pltpu.VMEM_SHARED((NC_ROWS + 16, H), jnp.float32),
        pltpu.VMEM((16,), jnp.int32),
        pltpu.VMEM((16,), jnp.int32),
        pltpu.VMEM((PBLK,), jnp.int32),
        pltpu.VMEM((PBLK,), jnp.int32),
        pltpu.VMEM((PBLK,), jnp.int32),
        pltpu.VMEM((PBLK,), jnp.int32),
        pltpu.VMEM((PBLK,), jnp.float32),
        pltpu.VMEM((PBLK,), jnp.float32),
        pltpu.VMEM((PBLK,), jnp.int32),
        pltpu.VMEM((PBLK,), jnp.int32),
        pltpu.VMEM((PBLK, H), jnp.float32),
        pltpu.VMEM((PBLK, H), jnp.float32),
        pltpu.VMEM((H,), jnp.float32),
        pltpu.VMEM((H,), jnp.float32),
        pltpu.SemaphoreType.DMA,
        pltpu.SemaphoreType.DMA,
    ],
)
def _sc_message(x, s2, d2, a2, st, ln, w, be, agg,
                agg_sp, stv, lnv, gatiA, gatdA, gatiB, gatdB,
                ablkA, ablkB, dblk, sblk, xgA, xgB, wv, bev, semA, semB):
    _msg_body(x, s2, d2, a2, st, ln, w, be, agg,
              agg_sp, stv, lnv, gatiA, gatdA, gatiB, gatdB,
              ablkA, ablkB, dblk, sblk, xgA, xgB, wv, bev, semA, semB)


def _proj_body(h_ref, w_ref, b_ref, o_ref):
    o_ref[...] = jnp.maximum(
        jnp.dot(h_ref[...], w_ref[...], preferred_element_type=jnp.float32)
        + b_ref[...], 0.0)


def _conv_body(x_ref, agg_ref, wn_ref, bn_ref, y_ref, cs_ref, ss_ref):
    i = pl.program_id(0)
    t = x_ref[...] + agg_ref[...]
    y = jnp.maximum(
        jnp.dot(t, wn_ref[...], preferred_element_type=jnp.float32)
        + bn_ref[...], 0.0)
    y_ref[...] = y
    rows = i * BLK + lax.broadcasted_iota(jnp.int32, (BLK, 1), 0)
    ym = jnp.where(rows < N, y, 0.0)

    @pl.when(i == 0)
    def _():
        cs_ref[...] = jnp.zeros_like(cs_ref)
        ss_ref[...] = jnp.zeros_like(ss_ref)

    cs_ref[...] += jnp.sum(ym, axis=0, keepdims=True)
    ss_ref[...] += jnp.sum(ym * ym, axis=0, keepdims=True)


def _norm_body(y_ref, cs_ref, ss_ref, o_ref):
    mu = cs_ref[...] / float(N)
    var = jnp.sum(ss_ref[...]) / float(N) - jnp.sum(mu * mu)
    denom = 1e-5 + jnp.sqrt(jnp.maximum(var, 0.0))
    o_ref[...] = (y_ref[...] - mu) / denom


def _pool_body(y_ref, bt_ref, w1_ref, b1_ref, w2_ref, b2_ref, o_ref, hg_ref):
    i = pl.program_id(0)

    @pl.when(i == 0)
    def _():
        hg_ref[...] = jnp.full_like(hg_ref, -jnp.inf)

    bt = bt_ref[...].reshape(BLK, 1)
    y = y_ref[...]
    glo = jnp.min(bt)
    ghi = jnp.minimum(jnp.max(bt), G - 1)

    def gbody(g, _):
        m = bt == g
        contrib = jnp.max(jnp.where(m, y, -jnp.inf), axis=0, keepdims=True)
        hg_ref[pl.ds(g, 1), :] = jnp.maximum(hg_ref[pl.ds(g, 1), :], contrib)
        return 0

    lax.fori_loop(glo, ghi + 1, gbody, 0)

    @pl.when(i == NBLOCKS - 1)
    def _():
        t = jnp.maximum(
            jnp.dot(hg_ref[...], w1_ref[...],
                    preferred_element_type=jnp.float32) + b1_ref[...], 0.0)
        o_ref[...] = jnp.dot(t, w2_ref[...],
                             preferred_element_type=jnp.float32) + b2_ref[...]


def _full(shape):
    return pl.BlockSpec(shape, lambda i: tuple(0 for _ in shape))


def kernel(h, edge_index, edge_attr, batch, W_in, b_in,
           We1, be1, Wn1, bn1, We2, be2, Wn2, bn2, We3, be3, Wn3, bn3,
           W1, b1, W2, b2):
    f32 = jnp.float32
    h_p = jnp.zeros((N_TC, 8), f32).at[:N, :5].set(h)
    W_inp = jnp.zeros((8, H), f32).at[:5].set(W_in)
    src_p = jnp.concatenate([edge_index[0], jnp.zeros((E_PAD - E,), jnp.int32)])
    dst_p = jnp.concatenate([edge_index[1],
                             jnp.full((E_PAD - E,), N_TC - 1, jnp.int32)])
    a_p = jnp.concatenate([edge_attr[:, 0], jnp.zeros((E_PAD - E,), f32)])
    batch_r = jnp.concatenate(
        [batch, jnp.full((N_TC - N,), G, jnp.int32)]).reshape(NBLOCKS, BLK, 1)
    W2p = jnp.zeros((H, 128), f32).at[:, :2].set(W2)
    b2p = jnp.zeros((1, 128), f32).at[0, :2].set(b2)

    x = pl.pallas_call(
        _proj_body,
        grid=(NBLOCKS,),
        in_specs=[pl.BlockSpec((BLK, 8), lambda i: (i, 0)),
                  _full((8, H)), _full((1, H))],
        out_specs=pl.BlockSpec((BLK, H), lambda i: (i, 0)),
        out_shape=jax.ShapeDtypeStruct((N_TC, H), f32),
    )(h_p, W_inp, b_in.reshape(1, H))

    src2, dl2, a2, starts, lens = _partition(src_p, dst_p, a_p)

    for We, be, Wn, bn in ((We1, be1, Wn1, bn1), (We2, be2, Wn2, bn2),
                           (We3, be3, Wn3, bn3)):
        agg = _sc_message(x, src2, dl2, a2, starts, lens, We[0], be)
        y, cs, ss = pl.pallas_call(
            _conv_body,
            grid=(NBLOCKS,),
            in_specs=[pl.BlockSpec((BLK, H), lambda i: (i, 0)),
                      pl.BlockSpec((BLK, H), lambda i: (i, 0)),
                      _full((H, H)), _full((1, H))],
            out_specs=[pl.BlockSpec((BLK, H), lambda i: (i, 0)),
                       _full((1, H)), _full((1, H))],
            out_shape=[jax.ShapeDtypeStruct((N_TC, H), f32),
                       jax.ShapeDtypeStruct((1, H), f32),
                       jax.ShapeDtypeStruct((1, H), f32)],
        )(x, agg, Wn, bn.reshape(1, H))
        x = pl.pallas_call(
            _norm_body,
            grid=(NBLOCKS,),
            in_specs=[pl.BlockSpec((BLK, H), lambda i: (i, 0)),
                      _full((1, H)), _full((1, H))],
            out_specs=pl.BlockSpec((BLK, H), lambda i: (i, 0)),
            out_shape=jax.ShapeDtypeStruct((N_TC, H), f32),
        )(y, cs, ss)

    out = pl.pallas_call(
        _pool_body,
        grid=(NBLOCKS,),
        in_specs=[pl.BlockSpec((BLK, H), lambda i: (i, 0)),
                  pl.BlockSpec((1, BLK, 1), lambda i: (i, 0, 0)),
                  _full((H, H)), _full((1, H)), _full((H, 128)),
                  _full((1, 128))],
        out_specs=_full((G, 128)),
        out_shape=jax.ShapeDtypeStruct((G, 128), f32),
        scratch_shapes=[pltpu.VMEM((G, 128), f32)],
    )(x, batch_r, W1, b1.reshape(1, H), W2p, b2p)
    return out[:, :2]


# speedup vs baseline: 12.8173x; 1.1434x over previous
import functools

import jax
import jax.numpy as jnp
from jax import lax
from jax.experimental import pallas as pl
from jax.experimental.pallas import tpu as pltpu
from jax.experimental.pallas import tpu_sc as plsc

N = 100000
E = 1600000
H = 128
G = 64

N_TC = 100352
NBLOCKS = 98
BLK = 1024
NC_ROWS = 11904
STRIPE = 744
NCH = 9
NPASS = 5
AGG_ROWS = 107520
E_PAD = 1601536
REG_E = 50048
SLOT = 50176
E_PAD2 = 32 * SLOT + 640
PBLK = 128
NPBLK = REG_E // PBLK


def _chunk_of(d):
    ch = jnp.zeros((16,), jnp.int32)
    for k in range(1, NCH):
        ch = ch + jnp.where(d >= k * NC_ROWS, 1, 0)
    return ch


def _part_body(src_ref, dst_ref, a_ref, src2_ref, dl2_ref, a2_ref,
               starts_ref, lens_ref,
               dblk, sblk, ablk, stg_s, stg_d, stg_a, cvec,
               fill, done, sstart, sem):
    c = lax.axis_index("c")
    s = lax.axis_index("s")
    w = 2 * s + c
    bin_ = w * REG_E
    bout = w * SLOT
    iota16 = lax.iota(jnp.int32, 16)

    def cnt_blk(b, accs):
        pltpu.sync_copy(dst_ref.at[pl.ds(bin_ + PBLK * b, PBLK)], dblk)

        def cnt_vreg(v, acc2):
            ch = _chunk_of(dblk[pl.ds(16 * v, 16)])
            return tuple(acc2[cc] + jnp.where(ch == cc, 1, 0)
                         for cc in range(NCH))

        return lax.fori_loop(0, PBLK // 16, cnt_vreg, accs)

    accs = lax.fori_loop(
        0, NPBLK, cnt_blk,
        tuple(jnp.zeros((16,), jnp.int32) for _ in range(NCH)))

    run = jnp.int32(0)
    startvec = jnp.zeros((16,), jnp.int32)
    lenvec = jnp.zeros((16,), jnp.int32)
    for cc in range(NCH):
        tot = accs[cc][0]
        for k in range(1, 16):
            tot = tot + accs[cc][k]
        plc = jnp.bitwise_and(tot + 7, -8)
        startvec = jnp.where(iota16 == cc, run, startvec)
        lenvec = jnp.where(iota16 == cc, plc, lenvec)
        sstart[cc] = run
        fill[cc] = jnp.int32(0)
        done[cc] = jnp.int32(0)
        run = run + plc
    cvec[...] = startvec
    pltpu.sync_copy(cvec, starts_ref.at[w])
    cvec[...] = lenvec
    pltpu.sync_copy(cvec, lens_ref.at[w])

    def sc_blk(b, _):
        off = bin_ + PBLK * b
        pltpu.sync_copy(dst_ref.at[pl.ds(off, PBLK)], dblk)
        pltpu.sync_copy(src_ref.at[pl.ds(off, PBLK)], sblk)
        pltpu.sync_copy(a_ref.at[pl.ds(off, PBLK)], ablk)

        def sc_vreg(v, _2):
            d16 = dblk[pl.ds(16 * v, 16)]
            s16 = sblk[pl.ds(16 * v, 16)]
            a16 = ablk[pl.ds(16 * v, 16)]
            ch16 = _chunk_of(d16)
            dl16 = d16 - ch16 * NC_ROWS
            for k in range(16):
                ck = ch16[k]
                n = fill[ck]
                al = jnp.bitwise_and(n, -16)
                msel = iota16 == (n - al)
                vd = stg_d[ck, pl.ds(al, 16)]
                stg_d[ck, pl.ds(al, 16)] = jnp.where(msel, dl16[k], vd)
                vs = stg_s[ck, pl.ds(al, 16)]
                stg_s[ck, pl.ds(al, 16)] = jnp.where(msel, s16[k], vs)
                va = stg_a[ck, pl.ds(al, 16)]
                stg_a[ck, pl.ds(al, 16)] = jnp.where(msel, a16[k], va)

                def flsh(z, ck=ck, n=n):
                    pos = pl.multiple_of(bout + sstart[ck] + done[ck], 8)
                    pltpu.sync_copy(stg_d.at[ck], dl2_ref.at[pl.ds(pos, 64)])
                    pltpu.sync_copy(stg_s.at[ck], src2_ref.at[pl.ds(pos, 64)])
                    pltpu.sync_copy(stg_a.at[ck], a2_ref.at[pl.ds(pos, 64)])
                    done[ck] = done[ck] + 64
                    fill[ck] = jnp.int32(0)
                    return z

                def noflsh(z, ck=ck, n=n):
                    fill[ck] = n + 1
                    return z

                lax.cond(n + 1 >= 64, flsh, noflsh, 0)
            return 0

        lax.fori_loop(0, PBLK // 16, sc_vreg, 0)
        return 0

    lax.fori_loop(0, NPBLK, sc_blk, 0)

    for ck in range(NCH):
        def pad1(q, _, ck=ck):
            n = fill[ck]
            al = jnp.bitwise_and(n, -16)
            ph = n - al
            msel = iota16 == ph
            vd = stg_d[ck, pl.ds(al, 16)]
            stg_d[ck, pl.ds(al, 16)] = jnp.where(msel, NC_ROWS + ph, vd)
            vs = stg_s[ck, pl.ds(al, 16)]
            stg_s[ck, pl.ds(al, 16)] = jnp.where(msel, ph, vs)
            va = stg_a[ck, pl.ds(al, 16)]
            stg_a[ck, pl.ds(al, 16)] = jnp.where(msel, 0.0, va)
            fill[ck] = n + 1
            return 0

        lax.fori_loop(0, jnp.bitwise_and(-fill[ck], 7), pad1, 0)

        def fl8(q, _, ck=ck):
            o8 = 8 * q
            pos = pl.multiple_of(bout + sstart[ck] + done[ck] + o8, 8)
            pltpu.sync_copy(stg_d.at[ck, pl.ds(o8, 8)],
                            dl2_ref.at[pl.ds(pos, 8)])
            pltpu.sync_copy(stg_s.at[ck, pl.ds(o8, 8)],
                            src2_ref.at[pl.ds(pos, 8)])
            pltpu.sync_copy(stg_a.at[ck, pl.ds(o8, 8)],
                            a2_ref.at[pl.ds(pos, 8)])
            return 0

        lax.fori_loop(0, fill[ck] // 8, fl8, 0)


@functools.partial(
    pl.kernel,
    out_type=(jax.ShapeDtypeStruct((E_PAD2,), jnp.int32),
              jax.ShapeDtypeStruct((E_PAD2,), jnp.int32),
              jax.ShapeDtypeStruct((E_PAD2,), jnp.float32),
              jax.ShapeDtypeStruct((32, 16), jnp.int32),
              jax.ShapeDtypeStruct((32, 16), jnp.int32)),
    mesh=plsc.VectorSubcoreMesh(core_axis_name="c", subcore_axis_name="s"),
    scratch_types=[
        pltpu.VMEM((PBLK,), jnp.int32),
        pltpu.VMEM((PBLK,), jnp.int32),
        pltpu.VMEM((PBLK,), jnp.float32),
        pltpu.VMEM((NCH, 64), jnp.int32),
        pltpu.VMEM((NCH, 64), jnp.int32),
        pltpu.VMEM((NCH, 64), jnp.float32),
        pltpu.VMEM((16,), jnp.int32),
        pltpu.SMEM((16,), jnp.int32),
        pltpu.SMEM((16,), jnp.int32),
        pltpu.SMEM((16,), jnp.int32),
        pltpu.SemaphoreType.DMA,
    ],
)
def _partition(src, dst, a, src2, dl2, a2, starts, lens,
               dblk, sblk, ablk, stg_s, stg_d, stg_a, cvec,
               fill, done, sstart, sem):
    _part_body(src, dst, a, src2, dl2, a2, starts, lens,
               dblk, sblk, ablk, stg_s, stg_d, stg_a, cvec,
               fill, done, sstart, sem)


def _msg_body(x_ref, s2_ref, d2_ref, a2_ref, st_ref, ln_ref, w_ref, be_ref,
              agg_ref, agg_sp, stv, lnv, gatiA, gatdA, gatiB, gatdB,
              ablkA, ablkB, dblk, sblk, xgA, xgB, wv, bev, semA, semB,
              semP, semSA, semSB):
    c = lax.axis_index("c")
    s = lax.axis_index("s")
    pltpu.sync_copy(w_ref, wv)
    pltpu.sync_copy(be_ref, bev)
    iota16 = lax.iota(jnp.int32, 16)
    wregs = [wv[pl.ds(16 * j, 16)] for j in range(8)]
    bregs = [bev[pl.ds(16 * j, 16)] for j in range(8)]

    def mk_rows(xg, ablk):
        def row_body(g, _):
            a16 = ablk[pl.ds(16 * g, 16)]
            for k in range(16):
                i = 16 * g + k
                asp = jnp.full((16,), a16[k])
                for j in range(8):
                    xv = xg[i, pl.ds(16 * j, 16)]
                    xg[i, pl.ds(16 * j, 16)] = jnp.maximum(
                        xv + asp * wregs[j] + bregs[j], 0.0)
            return 0
        return row_body

    rowA = mk_rows(xgA, ablkA)
    rowB = mk_rows(xgB, ablkB)

    def prep(boff, seg0, seglen, gi, gd, ab):
        off = pl.multiple_of(seg0 + PBLK * boff, 8)
        c1 = pltpu.async_copy(d2_ref.at[pl.ds(off, PBLK)], dblk, semP)
        c2 = pltpu.async_copy(s2_ref.at[pl.ds(off, PBLK)], sblk, semP)
        c3 = pltpu.async_copy(a2_ref.at[pl.ds(off, PBLK)], ab, semP)
        c1.wait()
        c2.wait()
        c3.wait()
        lpos = PBLK * boff
        for v in range(PBLK // 16):
            msk = (lpos + 16 * v + iota16) < seglen
            gd[pl.ds(16 * v, 16)] = jnp.where(
                msk, dblk[pl.ds(16 * v, 16)], NC_ROWS + iota16)
            gi[pl.ds(16 * v, 16)] = jnp.where(
                msk, sblk[pl.ds(16 * v, 16)], iota16)

    def seg_process(seg0, seglen):
        npair = (seglen + 2 * PBLK - 1) // (2 * PBLK)
        prep(0, seg0, seglen, gatiA, gatdA, ablkA)
        pltpu.async_copy(x_ref.at[gatiA], xgA, semA)
        prep(1, seg0, seglen, gatiB, gatdB, ablkB)
        pltpu.async_copy(x_ref.at[gatiB], xgB, semB)

        def pair(k2, _):
            b = 2 * k2
            pltpu.make_async_copy(x_ref.at[gatiA], xgA, semA).wait()
            lax.fori_loop(0, PBLK // 16, rowA, 0)
            pltpu.async_copy(xgA, agg_sp.at[gatdA], semSA, add=True)
            pltpu.make_async_copy(x_ref.at[gatiB], xgB, semB).wait()
            lax.fori_loop(0, PBLK // 16, rowB, 0)
            pltpu.async_copy(xgB, agg_sp.at[gatdB], semSB, add=True)
            pltpu.make_async_copy(xgA, agg_sp.at[gatdA], semSA).wait()
            prep(b + 2, seg0, seglen, gatiA, gatdA, ablkA)
            pltpu.async_copy(x_ref.at[gatiA], xgA, semA)
            pltpu.make_async_copy(xgB, agg_sp.at[gatdB], semSB).wait()
            prep(b + 3, seg0, seglen, gatiB, gatdB, ablkB)
            pltpu.async_copy(x_ref.at[gatiB], xgB, semB)
            return 0

        lax.fori_loop(0, npair, pair, 0)
        pltpu.make_async_copy(x_ref.at[gatiA], xgA, semA).wait()
        pltpu.make_async_copy(x_ref.at[gatiB], xgB, semB).wait()

    def zero_row(i, _):
        for j in range(8):
            xgA[i, pl.ds(16 * j, 16)] = jnp.zeros((16,), jnp.float32)
        return 0

    def pass_body(p, _):
        chunk = 2 * p + c
        valid = chunk <= NCH - 1
        r0 = s * STRIPE

        @pl.when(valid)
        def _():
            lax.fori_loop(0, PBLK, zero_row, 0)
            for k in range(5):
                pltpu.sync_copy(xgA, agg_sp.at[pl.ds(r0 + 128 * k, 128)])
            pltpu.sync_copy(xgA.at[pl.ds(0, 104)],
                            agg_sp.at[pl.ds(r0 + 640, 104)])

        plsc.subcore_barrier()

        @pl.when(valid)
        def _():
            def reg_body(r, _2):
                w2 = 2 * s + r
                pltpu.sync_copy(st_ref.at[w2], stv)
                pltpu.sync_copy(ln_ref.at[w2], lnv)
                svec = stv[pl.ds(0, 16)]
                lvec = lnv[pl.ds(0, 16)]
                seg_start = jnp.int32(0)
                seg_len = jnp.int32(0)
                for q in range(NCH):
                    seg_start = jnp.where(chunk == q, svec[q], seg_start)
                    seg_len = jnp.where(chunk == q, lvec[q], seg_len)
                seg_process(pl.multiple_of(w2 * SLOT + seg_start, 8),
                            seg_len)
                return 0

            lax.fori_loop(0, 2, reg_body, 0)

        plsc.subcore_barrier()

        @pl.when(valid)
        def _():
            g0 = chunk * NC_ROWS + r0
            for k in range(5):
                pltpu.sync_copy(agg_sp.at[pl.ds(r0 + 128 * k, 128)],
                                agg_ref.at[pl.ds(g0 + 128 * k, 128)])
            pltpu.sync_copy(agg_sp.at[pl.ds(r0 + 640, 104)],
                            agg_ref.at[pl.ds(g0 + 640, 104)])

        plsc.subcore_barrier()
        return 0

    lax.fori_loop(0, NPASS, pass_body, 0)


@functools.partial(
    pl.kernel,
    out_type=jax.ShapeDtypeStruct((AGG_ROWS, H), jnp.float32),
    mesh=plsc.VectorSubcoreMesh(core_axis_name="c", subcore_axis_name="s"),
    scratch_types=[
        pltpu.VMEM_SHARED((NC_ROWS + 16, H), jnp.float32),
        pltpu.VMEM((16,), jnp.int32),
        pltpu.VMEM((16,), jnp.int32),
        pltpu.VMEM((PBLK,), jnp.int32),
        pltpu.VMEM((PBLK,), jnp.int32),
        pltpu.VMEM((PBLK,), jnp.int32),
        pltpu.VMEM((PBLK,), jnp.int32),
        pltpu.VMEM((PBLK,), jnp.float32),
        pltpu.VMEM((PBLK,), jnp.float32),
        pltpu.VMEM((PBLK,), jnp.int32),
        pltpu.VMEM((PBLK,), jnp.int32),
        pltpu.VMEM((PBLK, H), jnp.float32),
        pltpu.VMEM((PBLK, H), jnp.float32),
        pltpu.VMEM((H,), jnp.float32),
        pltpu.VMEM((H,), jnp.float32),
        pltpu.SemaphoreType.DMA,
        pltpu.SemaphoreType.DMA,
        pltpu.SemaphoreType.DMA,
        pltpu.SemaphoreType.DMA,
        pltpu.SemaphoreType.DMA,
    ],
)
def _sc_message(x, s2, d2, a2, st, ln, w, be, agg,
                agg_sp, stv, lnv, gatiA, gatdA, gatiB, gatdB,
                ablkA, ablkB, dblk, sblk, xgA, xgB, wv, bev, semA, semB,
                semP, semSA, semSB):
    _msg_body(x, s2, d2, a2, st, ln, w, be, agg,
              agg_sp, stv, lnv, gatiA, gatdA, gatiB, gatdB,
              ablkA, ablkB, dblk, sblk, xgA, xgB, wv, bev, semA, semB,
              semP, semSA, semSB)


def _proj_body(h_ref, w_ref, b_ref, o_ref):
    o_ref[...] = jnp.maximum(
        jnp.dot(h_ref[...], w_ref[...], preferred_element_type=jnp.float32)
        + b_ref[...], 0.0)


def _conv_body(x_ref, agg_ref, wn_ref, bn_ref, y_ref, cs_ref, ss_ref):
    i = pl.program_id(0)
    t = x_ref[...] + agg_ref[...]
    y = jnp.maximum(
        jnp.dot(t, wn_ref[...], preferred_element_type=jnp.float32)
        + bn_ref[...], 0.0)
    y_ref[...] = y
    rows = i * BLK + lax.broadcasted_iota(jnp.int32, (BLK, 1), 0)
    ym = jnp.where(rows < N, y, 0.0)

    @pl.when(i == 0)
    def _():
        cs_ref[...] = jnp.zeros_like(cs_ref)
        ss_ref[...] = jnp.zeros_like(ss_ref)

    cs_ref[...] += jnp.sum(ym, axis=0, keepdims=True)
    ss_ref[...] += jnp.sum(ym * ym, axis=0, keepdims=True)


def _norm_body(y_ref, cs_ref, ss_ref, o_ref):
    mu = cs_ref[...] / float(N)
    var = jnp.sum(ss_ref[...]) / float(N) - jnp.sum(mu * mu)
    denom = 1e-5 + jnp.sqrt(jnp.maximum(var, 0.0))
    o_ref[...] = (y_ref[...] - mu) / denom


def _pool_body(y_ref, bt_ref, w1_ref, b1_ref, w2_ref, b2_ref, o_ref, hg_ref):
    i = pl.program_id(0)

    @pl.when(i == 0)
    def _():
        hg_ref[...] = jnp.full_like(hg_ref, -jnp.inf)

    bt = bt_ref[...].reshape(BLK, 1)
    y = y_ref[...]
    glo = jnp.min(bt)
    ghi = jnp.minimum(jnp.max(bt), G - 1)

    def gbody(g, _):
        m = bt == g
        contrib = jnp.max(jnp.where(m, y, -jnp.inf), axis=0, keepdims=True)
        hg_ref[pl.ds(g, 1), :] = jnp.maximum(hg_ref[pl.ds(g, 1), :], contrib)
        return 0

    lax.fori_loop(glo, ghi + 1, gbody, 0)

    @pl.when(i == NBLOCKS - 1)
    def _():
        t = jnp.maximum(
            jnp.dot(hg_ref[...], w1_ref[...],
                    preferred_element_type=jnp.float32) + b1_ref[...], 0.0)
        o_ref[...] = jnp.dot(t, w2_ref[...],
                             preferred_element_type=jnp.float32) + b2_ref[...]


def _full(shape):
    return pl.BlockSpec(shape, lambda i: tuple(0 for _ in shape))


def kernel(h, edge_index, edge_attr, batch, W_in, b_in,
           We1, be1, Wn1, bn1, We2, be2, Wn2, bn2, We3, be3, Wn3, bn3,
           W1, b1, W2, b2):
    f32 = jnp.float32
    h_p = jnp.zeros((N_TC, 8), f32).at[:N, :5].set(h)
    W_inp = jnp.zeros((8, H), f32).at[:5].set(W_in)
    src_p = jnp.concatenate([edge_index[0], jnp.zeros((E_PAD - E,), jnp.int32)])
    dst_p = jnp.concatenate([edge_index[1],
                             jnp.full((E_PAD - E,), N_TC - 1, jnp.int32)])
    a_p = jnp.concatenate([edge_attr[:, 0], jnp.zeros((E_PAD - E,), f32)])
    batch_r = jnp.concatenate(
        [batch, jnp.full((N_TC - N,), G, jnp.int32)]).reshape(NBLOCKS, BLK, 1)
    W2p = jnp.zeros((H, 128), f32).at[:, :2].set(W2)
    b2p = jnp.zeros((1, 128), f32).at[0, :2].set(b2)

    x = pl.pallas_call(
        _proj_body,
        grid=(NBLOCKS,),
        in_specs=[pl.BlockSpec((BLK, 8), lambda i: (i, 0)),
                  _full((8, H)), _full((1, H))],
        out_specs=pl.BlockSpec((BLK, H), lambda i: (i, 0)),
        out_shape=jax.ShapeDtypeStruct((N_TC, H), f32),
    )(h_p, W_inp, b_in.reshape(1, H))

    src2, dl2, a2, starts, lens = _partition(src_p, dst_p, a_p)

    for We, be, Wn, bn in ((We1, be1, Wn1, bn1), (We2, be2, Wn2, bn2),
                           (We3, be3, Wn3, bn3)):
        agg = _sc_message(x, src2, dl2, a2, starts, lens, We[0], be)
        y, cs, ss = pl.pallas_call(
            _conv_body,
            grid=(NBLOCKS,),
            in_specs=[pl.BlockSpec((BLK, H), lambda i: (i, 0)),
                      pl.BlockSpec((BLK, H), lambda i: (i, 0)),
                      _full((H, H)), _full((1, H))],
            out_specs=[pl.BlockSpec((BLK, H), lambda i: (i, 0)),
                       _full((1, H)), _full((1, H))],
            out_shape=[jax.ShapeDtypeStruct((N_TC, H), f32),
                       jax.ShapeDtypeStruct((1, H), f32),
                       jax.ShapeDtypeStruct((1, H), f32)],
        )(x, agg, Wn, bn.reshape(1, H))
        x = pl.pallas_call(
            _norm_body,
            grid=(NBLOCKS,),
            in_specs=[pl.BlockSpec((BLK, H), lambda i: (i, 0)),
                      _full((1, H)), _full((1, H))],
            out_specs=pl.BlockSpec((BLK, H), lambda i: (i, 0)),
            out_shape=jax.ShapeDtypeStruct((N_TC, H), f32),
        )(y, cs, ss)

    out = pl.pallas_call(
        _pool_body,
        grid=(NBLOCKS,),
        in_specs=[pl.BlockSpec((BLK, H), lambda i: (i, 0)),
                  pl.BlockSpec((1, BLK, 1), lambda i: (i, 0, 0)),
                  _full((H, H)), _full((1, H)), _full((H, 128)),
                  _full((1, 128))],
        out_specs=_full((G, 128)),
        out_shape=jax.ShapeDtypeStruct((G, 128), f32),
        scratch_shapes=[pltpu.VMEM((G, 128), f32)],
    )(x, batch_r, W1, b1.reshape(1, H), W2p, b2p)
    return out[:, :2]


# speedup vs baseline: 12.8808x; 1.0050x over previous
import functools

import jax
import jax.numpy as jnp
from jax import lax
from jax.experimental import pallas as pl
from jax.experimental.pallas import tpu as pltpu
from jax.experimental.pallas import tpu_sc as plsc

N = 100000
E = 1600000
H = 128
G = 64

N_TC = 100352
NBLOCKS = 98
BLK = 1024
NC_ROWS = 11904
STRIPE = 744
NCH = 9
NPASS = 5
AGG_ROWS = 107520
E_PAD = 1601536
REG_E = 50048
SLOT = 50176
E_PAD2 = 32 * SLOT + 640
PBLK = 128
NPBLK = REG_E // PBLK


def _chunk_of(d):
    ch = jnp.zeros((16,), jnp.int32)
    for k in range(1, NCH):
        ch = ch + jnp.where(d >= k * NC_ROWS, 1, 0)
    return ch


def _part_body(src_ref, dst_ref, a_ref, src2_ref, dl2_ref, a2_ref,
               starts_ref, lens_ref,
               dblk, sblk, ablk, stg_s, stg_d, stg_a, cvec,
               fill, done, sstart, sem):
    c = lax.axis_index("c")
    s = lax.axis_index("s")
    w = 2 * s + c
    bin_ = w * REG_E
    bout = w * SLOT
    iota16 = lax.iota(jnp.int32, 16)

    def cnt_blk(b, accs):
        pltpu.sync_copy(dst_ref.at[pl.ds(bin_ + PBLK * b, PBLK)], dblk)

        def cnt_vreg(v, acc2):
            ch = _chunk_of(dblk[pl.ds(16 * v, 16)])
            return tuple(acc2[cc] + jnp.where(ch == cc, 1, 0)
                         for cc in range(NCH))

        return lax.fori_loop(0, PBLK // 16, cnt_vreg, accs)

    accs = lax.fori_loop(
        0, NPBLK, cnt_blk,
        tuple(jnp.zeros((16,), jnp.int32) for _ in range(NCH)))

    run = jnp.int32(0)
    startvec = jnp.zeros((16,), jnp.int32)
    lenvec = jnp.zeros((16,), jnp.int32)
    for cc in range(NCH):
        tot = accs[cc][0]
        for k in range(1, 16):
            tot = tot + accs[cc][k]
        plc = jnp.bitwise_and(tot + 7, -8)
        startvec = jnp.where(iota16 == cc, run, startvec)
        lenvec = jnp.where(iota16 == cc, plc, lenvec)
        sstart[cc] = run
        fill[cc] = jnp.int32(0)
        done[cc] = jnp.int32(0)
        run = run + plc
    cvec[...] = startvec
    pltpu.sync_copy(cvec, starts_ref.at[w])
    cvec[...] = lenvec
    pltpu.sync_copy(cvec, lens_ref.at[w])

    def sc_blk(b, _):
        off = bin_ + PBLK * b
        pltpu.sync_copy(dst_ref.at[pl.ds(off, PBLK)], dblk)
        pltpu.sync_copy(src_ref.at[pl.ds(off, PBLK)], sblk)
        pltpu.sync_copy(a_ref.at[pl.ds(off, PBLK)], ablk)

        def sc_vreg(v, _2):
            d16 = dblk[pl.ds(16 * v, 16)]
            s16 = sblk[pl.ds(16 * v, 16)]
            a16 = ablk[pl.ds(16 * v, 16)]
            ch16 = _chunk_of(d16)
            dl16 = d16 - ch16 * NC_ROWS
            for k in range(16):
                ck = ch16[k]
                n = fill[ck]
                al = jnp.bitwise_and(n, -16)
                msel = iota16 == (n - al)
                vd = stg_d[ck, pl.ds(al, 16)]
                stg_d[ck, pl.ds(al, 16)] = jnp.where(msel, dl16[k], vd)
                vs = stg_s[ck, pl.ds(al, 16)]
                stg_s[ck, pl.ds(al, 16)] = jnp.where(msel, s16[k], vs)
                va = stg_a[ck, pl.ds(al, 16)]
                stg_a[ck, pl.ds(al, 16)] = jnp.where(msel, a16[k], va)

                def flsh(z, ck=ck, n=n):
                    pos = pl.multiple_of(bout + sstart[ck] + done[ck], 8)
                    f1 = pltpu.async_copy(stg_d.at[ck],
                                          dl2_ref.at[pl.ds(pos, 64)], sem)
                    f2 = pltpu.async_copy(stg_s.at[ck],
                                          src2_ref.at[pl.ds(pos, 64)], sem)
                    f3 = pltpu.async_copy(stg_a.at[ck],
                                          a2_ref.at[pl.ds(pos, 64)], sem)
                    f1.wait()
                    f2.wait()
                    f3.wait()
                    done[ck] = done[ck] + 64
                    fill[ck] = jnp.int32(0)
                    return z

                def noflsh(z, ck=ck, n=n):
                    fill[ck] = n + 1
                    return z

                lax.cond(n + 1 >= 64, flsh, noflsh, 0)
            return 0

        lax.fori_loop(0, PBLK // 16, sc_vreg, 0)
        return 0

    lax.fori_loop(0, NPBLK, sc_blk, 0)

    for ck in range(NCH):
        def pad1(q, _, ck=ck):
            n = fill[ck]
            al = jnp.bitwise_and(n, -16)
            ph = n - al
            msel = iota16 == ph
            vd = stg_d[ck, pl.ds(al, 16)]
            stg_d[ck, pl.ds(al, 16)] = jnp.where(msel, NC_ROWS + ph, vd)
            vs = stg_s[ck, pl.ds(al, 16)]
            stg_s[ck, pl.ds(al, 16)] = jnp.where(msel, ph, vs)
            va = stg_a[ck, pl.ds(al, 16)]
            stg_a[ck, pl.ds(al, 16)] = jnp.where(msel, 0.0, va)
            fill[ck] = n + 1
            return 0

        lax.fori_loop(0, jnp.bitwise_and(-fill[ck], 7), pad1, 0)

        def fl8(q, _, ck=ck):
            o8 = 8 * q
            pos = pl.multiple_of(bout + sstart[ck] + done[ck] + o8, 8)
            pltpu.sync_copy(stg_d.at[ck, pl.ds(o8, 8)],
                            dl2_ref.at[pl.ds(pos, 8)])
            pltpu.sync_copy(stg_s.at[ck, pl.ds(o8, 8)],
                            src2_ref.at[pl.ds(pos, 8)])
            pltpu.sync_copy(stg_a.at[ck, pl.ds(o8, 8)],
                            a2_ref.at[pl.ds(pos, 8)])
            return 0

        lax.fori_loop(0, fill[ck] // 8, fl8, 0)


@functools.partial(
    pl.kernel,
    out_type=(jax.ShapeDtypeStruct((E_PAD2,), jnp.int32),
              jax.ShapeDtypeStruct((E_PAD2,), jnp.int32),
              jax.ShapeDtypeStruct((E_PAD2,), jnp.float32),
              jax.ShapeDtypeStruct((32, 16), jnp.int32),
              jax.ShapeDtypeStruct((32, 16), jnp.int32)),
    mesh=plsc.VectorSubcoreMesh(core_axis_name="c", subcore_axis_name="s"),
    scratch_types=[
        pltpu.VMEM((PBLK,), jnp.int32),
        pltpu.VMEM((PBLK,), jnp.int32),
        pltpu.VMEM((PBLK,), jnp.float32),
        pltpu.VMEM((NCH, 64), jnp.int32),
        pltpu.VMEM((NCH, 64), jnp.int32),
        pltpu.VMEM((NCH, 64), jnp.float32),
        pltpu.VMEM((16,), jnp.int32),
        pltpu.SMEM((16,), jnp.int32),
        pltpu.SMEM((16,), jnp.int32),
        pltpu.SMEM((16,), jnp.int32),
        pltpu.SemaphoreType.DMA,
    ],
)
def _partition(src, dst, a, src2, dl2, a2, starts, lens,
               dblk, sblk, ablk, stg_s, stg_d, stg_a, cvec,
               fill, done, sstart, sem):
    _part_body(src, dst, a, src2, dl2, a2, starts, lens,
               dblk, sblk, ablk, stg_s, stg_d, stg_a, cvec,
               fill, done, sstart, sem)


def _msg_body(x_ref, s2_ref, d2_ref, a2_ref, st_ref, ln_ref, w_ref, be_ref,
              agg_ref, agg_sp, stv, lnv, gatiA, gatdA, gatiB, gatdB,
              ablkA, ablkB, dblk, sblk, xgA, xgB, wv, bev, semA, semB,
              semP, semSA, semSB):
    c = lax.axis_index("c")
    s = lax.axis_index("s")
    pltpu.sync_copy(w_ref, wv)
    pltpu.sync_copy(be_ref, bev)
    iota16 = lax.iota(jnp.int32, 16)
    wregs = [wv[pl.ds(16 * j, 16)] for j in range(8)]
    bregs = [bev[pl.ds(16 * j, 16)] for j in range(8)]

    def mk_rows(xg, ablk):
        def row_body(g, _):
            a16 = ablk[pl.ds(16 * g, 16)]
            for k in range(16):
                i = 16 * g + k
                asp = jnp.full((16,), a16[k])
                for j in range(8):
                    xv = xg[i, pl.ds(16 * j, 16)]
                    xg[i, pl.ds(16 * j, 16)] = jnp.maximum(
                        xv + asp * wregs[j] + bregs[j], 0.0)
            return 0
        return row_body

    rowA = mk_rows(xgA, ablkA)
    rowB = mk_rows(xgB, ablkB)

    def prep(boff, seg0, seglen, gi, gd, ab):
        off = pl.multiple_of(seg0 + PBLK * boff, 8)
        c1 = pltpu.async_copy(d2_ref.at[pl.ds(off, PBLK)], dblk, semP)
        c2 = pltpu.async_copy(s2_ref.at[pl.ds(off, PBLK)], sblk, semP)
        c3 = pltpu.async_copy(a2_ref.at[pl.ds(off, PBLK)], ab, semP)
        c1.wait()
        c2.wait()
        c3.wait()
        lpos = PBLK * boff
        for v in range(PBLK // 16):
            msk = (lpos + 16 * v + iota16) < seglen
            gd[pl.ds(16 * v, 16)] = jnp.where(
                msk, dblk[pl.ds(16 * v, 16)], NC_ROWS + iota16)
            gi[pl.ds(16 * v, 16)] = jnp.where(
                msk, sblk[pl.ds(16 * v, 16)], iota16)

    def seg_process(seg0, seglen):
        npair = (seglen + 2 * PBLK - 1) // (2 * PBLK)
        prep(0, seg0, seglen, gatiA, gatdA, ablkA)
        pltpu.async_copy(x_ref.at[gatiA], xgA, semA)
        prep(1, seg0, seglen, gatiB, gatdB, ablkB)
        pltpu.async_copy(x_ref.at[gatiB], xgB, semB)

        def pair(k2, _):
            b = 2 * k2
            pltpu.make_async_copy(x_ref.at[gatiA], xgA, semA).wait()
            lax.fori_loop(0, PBLK // 16, rowA, 0)
            pltpu.async_copy(xgA, agg_sp.at[gatdA], semSA, add=True)
            pltpu.make_async_copy(x_ref.at[gatiB], xgB, semB).wait()
            lax.fori_loop(0, PBLK // 16, rowB, 0)
            pltpu.async_copy(xgB, agg_sp.at[gatdB], semSB, add=True)
            pltpu.make_async_copy(xgA, agg_sp.at[gatdA], semSA).wait()
            prep(b + 2, seg0, seglen, gatiA, gatdA, ablkA)
            pltpu.async_copy(x_ref.at[gatiA], xgA, semA)
            pltpu.make_async_copy(xgB, agg_sp.at[gatdB], semSB).wait()
            prep(b + 3, seg0, seglen, gatiB, gatdB, ablkB)
            pltpu.async_copy(x_ref.at[gatiB], xgB, semB)
            return 0

        lax.fori_loop(0, npair, pair, 0)
        pltpu.make_async_copy(x_ref.at[gatiA], xgA, semA).wait()
        pltpu.make_async_copy(x_ref.at[gatiB], xgB, semB).wait()

    def zero_row(i, _):
        for j in range(8):
            xgA[i, pl.ds(16 * j, 16)] = jnp.zeros((16,), jnp.float32)
        return 0

    def pass_body(p, _):
        chunk = 2 * p + c
        valid = chunk <= NCH - 1
        r0 = s * STRIPE

        @pl.when(valid)
        def _():
            lax.fori_loop(0, PBLK, zero_row, 0)
            for k in range(5):
                pltpu.sync_copy(xgA, agg_sp.at[pl.ds(r0 + 128 * k, 128)])
            pltpu.sync_copy(xgA.at[pl.ds(0, 104)],
                            agg_sp.at[pl.ds(r0 + 640, 104)])

        plsc.subcore_barrier()

        @pl.when(valid)
        def _():
            def reg_body(r, _2):
                w2 = 2 * s + r
                pltpu.sync_copy(st_ref.at[w2], stv)
                pltpu.sync_copy(ln_ref.at[w2], lnv)
                svec = stv[pl.ds(0, 16)]
                lvec = lnv[pl.ds(0, 16)]
                seg_start = jnp.int32(0)
                seg_len = jnp.int32(0)
                for q in range(NCH):
                    seg_start = jnp.where(chunk == q, svec[q], seg_start)
                    seg_len = jnp.where(chunk == q, lvec[q], seg_len)
                seg_process(pl.multiple_of(w2 * SLOT + seg_start, 8),
                            seg_len)
                return 0

            lax.fori_loop(0, 2, reg_body, 0)

        plsc.subcore_barrier()

        @pl.when(valid)
        def _():
            g0 = chunk * NC_ROWS + r0
            for k in range(5):
                pltpu.sync_copy(agg_sp.at[pl.ds(r0 + 128 * k, 128)],
                                agg_ref.at[pl.ds(g0 + 128 * k, 128)])
            pltpu.sync_copy(agg_sp.at[pl.ds(r0 + 640, 104)],
                            agg_ref.at[pl.ds(g0 + 640, 104)])

        plsc.subcore_barrier()
        return 0

    lax.fori_loop(0, NPASS, pass_body, 0)


@functools.partial(
    pl.kernel,
    out_type=jax.ShapeDtypeStruct((AGG_ROWS, H), jnp.float32),
    mesh=plsc.VectorSubcoreMesh(core_axis_name="c", subcore_axis_name="s"),
    scratch_types=[
        pltpu.VMEM_SHARED((NC_ROWS + 16, H), jnp.float32),
        pltpu.VMEM((16,), jnp.int32),
        pltpu.VMEM((16,), jnp.int32),
        pltpu.VMEM((PBLK,), jnp.int32),
        pltpu.VMEM((PBLK,), jnp.int32),
        pltpu.VMEM((PBLK,), jnp.int32),
        pltpu.VMEM((PBLK,), jnp.int32),
        pltpu.VMEM((PBLK,), jnp.float32),
        pltpu.VMEM((PBLK,), jnp.float32),
        pltpu.VMEM((PBLK,), jnp.int32),
        pltpu.VMEM((PBLK,), jnp.int32),
        pltpu.VMEM((PBLK, H), jnp.float32),
        pltpu.VMEM((PBLK, H), jnp.float32),
        pltpu.VMEM((H,), jnp.float32),
        pltpu.VMEM((H,), jnp.float32),
        pltpu.SemaphoreType.DMA,
        pltpu.SemaphoreType.DMA,
        pltpu.SemaphoreType.DMA,
        pltpu.SemaphoreType.DMA,
        pltpu.SemaphoreType.DMA,
    ],
)
def _sc_message(x, s2, d2, a2, st, ln, w, be, agg,
                agg_sp, stv, lnv, gatiA, gatdA, gatiB, gatdB,
                ablkA, ablkB, dblk, sblk, xgA, xgB, wv, bev, semA, semB,
                semP, semSA, semSB):
    _msg_body(x, s2, d2, a2, st, ln, w, be, agg,
              agg_sp, stv, lnv, gatiA, gatdA, gatiB, gatdB,
              ablkA, ablkB, dblk, sblk, xgA, xgB, wv, bev, semA, semB,
              semP, semSA, semSB)


def _proj_body(h_ref, w_ref, b_ref, o_ref):
    o_ref[...] = jnp.maximum(
        jnp.dot(h_ref[...], w_ref[...], preferred_element_type=jnp.float32)
        + b_ref[...], 0.0)


def _conv_body(x_ref, agg_ref, wn_ref, bn_ref, y_ref, cs_ref, ss_ref):
    i = pl.program_id(0)
    t = x_ref[...] + agg_ref[...]
    y = jnp.maximum(
        jnp.dot(t, wn_ref[...], preferred_element_type=jnp.float32)
        + bn_ref[...], 0.0)
    y_ref[...] = y
    rows = i * BLK + lax.broadcasted_iota(jnp.int32, (BLK, 1), 0)
    ym = jnp.where(rows < N, y, 0.0)

    @pl.when(i == 0)
    def _():
        cs_ref[...] = jnp.zeros_like(cs_ref)
        ss_ref[...] = jnp.zeros_like(ss_ref)

    cs_ref[...] += jnp.sum(ym, axis=0, keepdims=True)
    ss_ref[...] += jnp.sum(ym * ym, axis=0, keepdims=True)


def _norm_body(y_ref, cs_ref, ss_ref, o_ref):
    mu = cs_ref[...] / float(N)
    var = jnp.sum(ss_ref[...]) / float(N) - jnp.sum(mu * mu)
    denom = 1e-5 + jnp.sqrt(jnp.maximum(var, 0.0))
    o_ref[...] = (y_ref[...] - mu) / denom


def _pool_body(y_ref, bt_ref, w1_ref, b1_ref, w2_ref, b2_ref, o_ref, hg_ref):
    i = pl.program_id(0)

    @pl.when(i == 0)
    def _():
        hg_ref[...] = jnp.full_like(hg_ref, -jnp.inf)

    bt = bt_ref[...].reshape(BLK, 1)
    y = y_ref[...]
    glo = jnp.min(bt)
    ghi = jnp.minimum(jnp.max(bt), G - 1)

    def gbody(g, _):
        m = bt == g
        contrib = jnp.max(jnp.where(m, y, -jnp.inf), axis=0, keepdims=True)
        hg_ref[pl.ds(g, 1), :] = jnp.maximum(hg_ref[pl.ds(g, 1), :], contrib)
        return 0

    lax.fori_loop(glo, ghi + 1, gbody, 0)

    @pl.when(i == NBLOCKS - 1)
    def _():
        t = jnp.maximum(
            jnp.dot(hg_ref[...], w1_ref[...],
                    preferred_element_type=jnp.float32) + b1_ref[...], 0.0)
        o_ref[...] = jnp.dot(t, w2_ref[...],
                             preferred_element_type=jnp.float32) + b2_ref[...]


def _full(shape):
    return pl.BlockSpec(shape, lambda i: tuple(0 for _ in shape))


def kernel(h, edge_index, edge_attr, batch, W_in, b_in,
           We1, be1, Wn1, bn1, We2, be2, Wn2, bn2, We3, be3, Wn3, bn3,
           W1, b1, W2, b2):
    f32 = jnp.float32
    h_p = jnp.zeros((N_TC, 8), f32).at[:N, :5].set(h)
    W_inp = jnp.zeros((8, H), f32).at[:5].set(W_in)
    src_p = jnp.concatenate([edge_index[0], jnp.zeros((E_PAD - E,), jnp.int32)])
    dst_p = jnp.concatenate([edge_index[1],
                             jnp.full((E_PAD - E,), N_TC - 1, jnp.int32)])
    a_p = jnp.concatenate([edge_attr[:, 0], jnp.zeros((E_PAD - E,), f32)])
    batch_r = jnp.concatenate(
        [batch, jnp.full((N_TC - N,), G, jnp.int32)]).reshape(NBLOCKS, BLK, 1)
    W2p = jnp.zeros((H, 128), f32).at[:, :2].set(W2)
    b2p = jnp.zeros((1, 128), f32).at[0, :2].set(b2)

    x = pl.pallas_call(
        _proj_body,
        grid=(NBLOCKS,),
        in_specs=[pl.BlockSpec((BLK, 8), lambda i: (i, 0)),
                  _full((8, H)), _full((1, H))],
        out_specs=pl.BlockSpec((BLK, H), lambda i: (i, 0)),
        out_shape=jax.ShapeDtypeStruct((N_TC, H), f32),
    )(h_p, W_inp, b_in.reshape(1, H))

    src2, dl2, a2, starts, lens = _partition(src_p, dst_p, a_p)

    for We, be, Wn, bn in ((We1, be1, Wn1, bn1), (We2, be2, Wn2, bn2),
                           (We3, be3, Wn3, bn3)):
        agg = _sc_message(x, src2, dl2, a2, starts, lens, We[0], be)
        y, cs, ss = pl.pallas_call(
            _conv_body,
            grid=(NBLOCKS,),
            in_specs=[pl.BlockSpec((BLK, H), lambda i: (i, 0)),
                      pl.BlockSpec((BLK, H), lambda i: (i, 0)),
                      _full((H, H)), _full((1, H))],
            out_specs=[pl.BlockSpec((BLK, H), lambda i: (i, 0)),
                       _full((1, H)), _full((1, H))],
            out_shape=[jax.ShapeDtypeStruct((N_TC, H), f32),
                       jax.ShapeDtypeStruct((1, H), f32),
                       jax.ShapeDtypeStruct((1, H), f32)],
        )(x, agg, Wn, bn.reshape(1, H))
        x = pl.pallas_call(
            _norm_body,
            grid=(NBLOCKS,),
            in_specs=[pl.BlockSpec((BLK, H), lambda i: (i, 0)),
                      _full((1, H)), _full((1, H))],
            out_specs=pl.BlockSpec((BLK, H), lambda i: (i, 0)),
            out_shape=jax.ShapeDtypeStruct((N_TC, H), f32),
        )(y, cs, ss)

    out = pl.pallas_call(
        _pool_body,
        grid=(NBLOCKS,),
        in_specs=[pl.BlockSpec((BLK, H), lambda i: (i, 0)),
                  pl.BlockSpec((1, BLK, 1), lambda i: (i, 0, 0)),
                  _full((H, H)), _full((1, H)), _full((H, 128)),
                  _full((1, 128))],
        out_specs=_full((G, 128)),
        out_shape=jax.ShapeDtypeStruct((G, 128), f32),
        scratch_shapes=[pltpu.VMEM((G, 128), f32)],
    )(x, batch_r, W1, b1.reshape(1, H), W2p, b2p)
    return out[:, :2]
